# baseline (speedup 1.0000x reference)
import functools

import jax
import jax.numpy as jnp
from jax import lax
from jax.experimental import pallas as pl
from jax.experimental.pallas import tpu as pltpu
from jax.experimental.pallas import tpu_sc as plsc


def _topk_kernel(a1_ref, n1_ref, a2_ref, n2_ref, idx_ref, w_ref, *, S):
    a1 = a1_ref[0]
    a2 = a2_ref[0]
    cross = jnp.dot(a1, a2.T, preferred_element_type=jnp.float32)
    work = (n1_ref[0] + n2_ref[0]) - 2.0 * cross
    tn = work.shape[0]
    iota = lax.broadcasted_iota(jnp.int32, (tn, S), 1)
    wsum = jnp.zeros((tn, 1), jnp.float32)
    wks = []
    for k in range(3):
        mk = jnp.min(work, axis=-1, keepdims=True)
        mask = work == mk
        jk = jnp.min(jnp.where(mask, iota, S), axis=-1,
                     keepdims=True).astype(jnp.float32)
        dk = jnp.maximum(jnp.sqrt(jnp.maximum(mk, 0.0)), 1e-10)
        wk = 1.0 / dk
        wsum = wsum + wk
        wks.append(wk)
        idx_ref[0, :, k:k + 1] = jk
        if k < 2:
            work = jnp.where(mask, jnp.inf, work)
    for k in range(3):
        w_ref[0, :, k:k + 1] = wks[k] / wsum


def _mlp_kernel(f1_ref, it_ref, w1a_ref, w1b_ref, b1_ref, w2_ref, b2_ref,
                out_ref):
    f1 = f1_ref[0]
    it = it_ref[0]
    h = jnp.dot(f1, w1a_ref[...], preferred_element_type=jnp.float32)
    h = h + lax.dot_general(it, w1b_ref[...], (((0,), (0,)), ((), ())),
                            preferred_element_type=jnp.float32)
    h = jnp.maximum(h + b1_ref[...], 0.0)
    o = jnp.dot(h, w2_ref[...], preferred_element_type=jnp.float32)
    out_ref[0] = jnp.maximum(o + b2_ref[...], 0.0)


def _sc_interp(feat2, idxT, wT):
    B, S, C2 = feat2.shape
    N = idxT.shape[-1]
    NC, NS = 2, 16
    NW = NC * NS
    per_w = (B * N) // NW
    parts = NW // B
    mesh = plsc.VectorSubcoreMesh(core_axis_name="c", subcore_axis_name="s",
                                  num_cores=NC, num_subcores=NS)

    @functools.partial(
        pl.kernel, mesh=mesh,
        compiler_params=pltpu.CompilerParams(needs_layout_passes=False),
        out_type=jax.ShapeDtypeStruct((B, C2, N), jnp.float32),
        scratch_types=[
            pltpu.VMEM((S, C2), jnp.float32),
            pltpu.VMEM((3, per_w), jnp.int32),
            pltpu.VMEM((3, per_w), jnp.float32),
            pltpu.VMEM((C2, 128), jnp.float32),
        ],
    )
    def k(feat2_hbm, idx_hbm, w_hbm, out_hbm, f2_v, idx_v, w_v, ob_v):
        wid = lax.axis_index("s") * NC + lax.axis_index("c")
        b = wid // parts
        base = (wid % parts) * per_w
        pltpu.sync_copy(feat2_hbm.at[b], f2_v)
        pltpu.sync_copy(idx_hbm.at[b, :, pl.ds(base, per_w)], idx_v)
        pltpu.sync_copy(w_hbm.at[b, :, pl.ds(base, per_w)], w_v)

        def block(g, _):
            p0 = g * 128

            def sub(s, _):
                q0 = p0 + s * 16
                r0 = idx_v[0, pl.ds(q0, 16)]
                r1 = idx_v[1, pl.ds(q0, 16)]
                r2 = idx_v[2, pl.ds(q0, 16)]
                w0 = w_v[0, pl.ds(q0, 16)]
                w1 = w_v[1, pl.ds(q0, 16)]
                w2 = w_v[2, pl.ds(q0, 16)]
                for c in range(C2):
                    cc = jnp.full((16,), c, jnp.int32)
                    g0 = plsc.load_gather(f2_v, [r0, cc])
                    g1 = plsc.load_gather(f2_v, [r1, cc])
                    g2 = plsc.load_gather(f2_v, [r2, cc])
                    ob_v[c, pl.ds(s * 16, 16)] = g0 * w0 + g1 * w1 + g2 * w2
                return 0

            lax.fori_loop(0, 8, sub, 0)
            pltpu.sync_copy(ob_v, out_hbm.at[b, :, pl.ds(base + p0, 128)])
            return 0

        lax.fori_loop(0, per_w // 128, block, 0)

    return k(feat2, idxT, wT)


@jax.jit
def kernel(xyz1, xyz2, feat1, feat2, W1, b1, W2, b2):
    B, N, _ = xyz1.shape
    S = xyz2.shape[1]
    C1 = feat1.shape[-1]
    TN = 4096

    n1 = jnp.sum(xyz1 * xyz1, axis=-1)[..., None]
    n2 = jnp.sum(xyz2 * xyz2, axis=-1)[:, None, :]
    a1 = jnp.concatenate([xyz1, jnp.zeros((B, N, 5), xyz1.dtype)], axis=-1)
    a2 = jnp.concatenate([xyz2, jnp.zeros((B, S, 5), xyz2.dtype)], axis=-1)

    idx_f, w = pl.pallas_call(
        functools.partial(_topk_kernel, S=S),
        grid=(B, N // TN),
        in_specs=[
            pl.BlockSpec((1, TN, 8), lambda b, n: (b, n, 0)),
            pl.BlockSpec((1, TN, 1), lambda b, n: (b, n, 0)),
            pl.BlockSpec((1, S, 8), lambda b, n: (b, 0, 0)),
            pl.BlockSpec((1, 1, S), lambda b, n: (b, 0, 0)),
        ],
        out_specs=[
            pl.BlockSpec((1, TN, 3), lambda b, n: (b, n, 0)),
            pl.BlockSpec((1, TN, 3), lambda b, n: (b, n, 0)),
        ],
        out_shape=[
            jax.ShapeDtypeStruct((B, N, 3), jnp.float32),
            jax.ShapeDtypeStruct((B, N, 3), jnp.float32),
        ],
    )(a1, n1, a2, n2)

    idxT = jnp.transpose(idx_f, (0, 2, 1)).astype(jnp.int32)
    wT = jnp.transpose(w, (0, 2, 1))

    interpT = _sc_interp(feat2, idxT, wT)

    W1a = W1[:C1]
    W1b = W1[C1:]
    b1r = b1.reshape(1, -1)
    b2r = b2.reshape(1, -1)
    out = pl.pallas_call(
        _mlp_kernel,
        grid=(B, N // TN),
        in_specs=[
            pl.BlockSpec((1, TN, C1), lambda b, n: (b, n, 0)),
            pl.BlockSpec((1, feat2.shape[-1], TN), lambda b, n: (b, 0, n)),
            pl.BlockSpec(W1a.shape, lambda b, n: (0, 0)),
            pl.BlockSpec(W1b.shape, lambda b, n: (0, 0)),
            pl.BlockSpec(b1r.shape, lambda b, n: (0, 0)),
            pl.BlockSpec(W2.shape, lambda b, n: (0, 0)),
            pl.BlockSpec(b2r.shape, lambda b, n: (0, 0)),
        ],
        out_specs=pl.BlockSpec((1, TN, W2.shape[-1]), lambda b, n: (b, n, 0)),
        out_shape=jax.ShapeDtypeStruct((B, N, W2.shape[-1]), jnp.float32),
    )(feat1, interpT, W1a, W1b, b1r, W2, b2r)
    return out


# speedup vs baseline: 2.3037x; 2.3037x over previous
import functools

import jax
import jax.numpy as jnp
from jax import lax
from jax.experimental import pallas as pl


def _fp_kernel(a1_ref, n1_ref, f1_ref, a2_ref, n2_ref, f2_ref, w1a_ref,
               w1b_ref, b1_ref, w2_ref, b2_ref, out_ref, *, S):
    a1 = a1_ref[0]
    a2 = a2_ref[0]
    cross = jnp.dot(a1, a2.T, preferred_element_type=jnp.float32)
    work = (n1_ref[0] + n2_ref[0]) - 2.0 * cross

    tn = work.shape[0]
    acc = jnp.zeros((tn, S), jnp.float32)
    wsum = jnp.zeros((tn, 1), jnp.float32)
    cnt = jnp.zeros((tn, 1), jnp.float32)
    for k in range(3):
        mk = jnp.min(work, axis=-1, keepdims=True)
        mask = work == mk
        m01 = jnp.where(mask, 1.0, 0.0)
        c = jnp.sum(m01, axis=-1, keepdims=True)
        gate = cnt < 2.5
        dk = jnp.maximum(jnp.sqrt(jnp.maximum(mk, 0.0)), 1e-10)
        wk = jnp.where(gate, 1.0 / dk, 0.0)
        acc = acc + m01 * wk
        wsum = wsum + wk * c
        cnt = cnt + c
        if k < 2:
            work = jnp.where(mask, jnp.inf, work)

    f2 = f2_ref[0]
    interp = jnp.dot(acc, f2, preferred_element_type=jnp.float32) / wsum
    f1 = f1_ref[0]
    h = jnp.dot(f1, w1a_ref[...], preferred_element_type=jnp.float32)
    h = h + jnp.dot(interp, w1b_ref[...], preferred_element_type=jnp.float32)
    h = jnp.maximum(h + b1_ref[...], 0.0)
    o = jnp.dot(h, w2_ref[...], preferred_element_type=jnp.float32)
    out_ref[0] = jnp.maximum(o + b2_ref[...], 0.0)


@jax.jit
def kernel(xyz1, xyz2, feat1, feat2, W1, b1, W2, b2):
    B, N, _ = xyz1.shape
    S = xyz2.shape[1]
    C1 = feat1.shape[-1]
    TN = 4096

    n1 = jnp.sum(xyz1 * xyz1, axis=-1)[..., None]
    n2 = jnp.sum(xyz2 * xyz2, axis=-1)[:, None, :]
    a1 = jnp.concatenate([xyz1, jnp.zeros((B, N, 5), xyz1.dtype)], axis=-1)
    a2 = jnp.concatenate([xyz2, jnp.zeros((B, S, 5), xyz2.dtype)], axis=-1)

    W1a = W1[:C1]
    W1b = W1[C1:]
    b1r = b1.reshape(1, -1)
    b2r = b2.reshape(1, -1)

    grid = (B, N // TN)
    out = pl.pallas_call(
        functools.partial(_fp_kernel, S=S),
        grid=grid,
        in_specs=[
            pl.BlockSpec((1, TN, 8), lambda b, n: (b, n, 0)),
            pl.BlockSpec((1, TN, 1), lambda b, n: (b, n, 0)),
            pl.BlockSpec((1, TN, C1), lambda b, n: (b, n, 0)),
            pl.BlockSpec((1, S, 8), lambda b, n: (b, 0, 0)),
            pl.BlockSpec((1, 1, S), lambda b, n: (b, 0, 0)),
            pl.BlockSpec((1, S, feat2.shape[-1]), lambda b, n: (b, 0, 0)),
            pl.BlockSpec(W1a.shape, lambda b, n: (0, 0)),
            pl.BlockSpec(W1b.shape, lambda b, n: (0, 0)),
            pl.BlockSpec(b1r.shape, lambda b, n: (0, 0)),
            pl.BlockSpec(W2.shape, lambda b, n: (0, 0)),
            pl.BlockSpec(b2r.shape, lambda b, n: (0, 0)),
        ],
        out_specs=pl.BlockSpec((1, TN, W2.shape[-1]), lambda b, n: (b, n, 0)),
        out_shape=jax.ShapeDtypeStruct((B, N, W2.shape[-1]), jnp.float32),
    )(a1, n1, feat1, a2, n2, feat2, W1a, W1b, b1r, W2, b2r)
    return out


# speedup vs baseline: 2.5085x; 1.0889x over previous
import functools

import jax
import jax.numpy as jnp
from jax import lax
from jax.experimental import pallas as pl


def _fp_kernel(a1_ref, n1_ref, f1_ref, a2_ref, n2_ref, f2_ref, w1a_ref,
               w1b_ref, b1_ref, w2_ref, b2_ref, out_ref, *, S):
    a1 = a1_ref[0]
    a2 = a2_ref[0]
    cross = jnp.dot(a1, a2.T, preferred_element_type=jnp.float32)
    work = (n1_ref[0] + n2_ref[0]) - 2.0 * cross

    tn = work.shape[0]
    acc = jnp.zeros((tn, S), jnp.float32)
    wsum = jnp.zeros((tn, 1), jnp.float32)
    cnt = jnp.zeros((tn, 1), jnp.float32)
    for k in range(3):
        mk = jnp.min(work, axis=-1, keepdims=True)
        mask = work == mk
        m01 = jnp.where(mask, 1.0, 0.0)
        c = jnp.sum(m01, axis=-1, keepdims=True)
        gate = cnt < 2.5
        wk = jnp.where(gate,
                       jnp.minimum(lax.rsqrt(jnp.maximum(mk, 0.0)), 1e10),
                       0.0)
        acc = acc + m01 * wk
        wsum = wsum + wk * c
        cnt = cnt + c
        if k < 2:
            work = jnp.where(mask, jnp.inf, work)

    f2 = f2_ref[0]
    interp = jnp.dot(acc, f2, preferred_element_type=jnp.float32) / wsum
    f1 = f1_ref[0]
    h = jnp.dot(f1, w1a_ref[...], preferred_element_type=jnp.float32)
    h = h + jnp.dot(interp, w1b_ref[...], preferred_element_type=jnp.float32)
    h = jnp.maximum(h + b1_ref[...], 0.0)
    o = jnp.dot(h, w2_ref[...], preferred_element_type=jnp.float32)
    out_ref[0] = jnp.maximum(o + b2_ref[...], 0.0)


@jax.jit
def kernel(xyz1, xyz2, feat1, feat2, W1, b1, W2, b2):
    B, N, _ = xyz1.shape
    S = xyz2.shape[1]
    C1 = feat1.shape[-1]
    TN = 4096

    n1 = jnp.sum(xyz1 * xyz1, axis=-1)[..., None]
    n2 = jnp.sum(xyz2 * xyz2, axis=-1)[:, None, :]
    a1 = jnp.concatenate([xyz1, jnp.zeros((B, N, 5), xyz1.dtype)], axis=-1)
    a2 = jnp.concatenate([xyz2, jnp.zeros((B, S, 5), xyz2.dtype)], axis=-1)

    W1a = W1[:C1]
    W1b = W1[C1:]
    b1r = b1.reshape(1, -1)
    b2r = b2.reshape(1, -1)

    grid = (B, N // TN)
    out = pl.pallas_call(
        functools.partial(_fp_kernel, S=S),
        grid=grid,
        in_specs=[
            pl.BlockSpec((1, TN, 8), lambda b, n: (b, n, 0)),
            pl.BlockSpec((1, TN, 1), lambda b, n: (b, n, 0)),
            pl.BlockSpec((1, TN, C1), lambda b, n: (b, n, 0)),
            pl.BlockSpec((1, S, 8), lambda b, n: (b, 0, 0)),
            pl.BlockSpec((1, 1, S), lambda b, n: (b, 0, 0)),
            pl.BlockSpec((1, S, feat2.shape[-1]), lambda b, n: (b, 0, 0)),
            pl.BlockSpec(W1a.shape, lambda b, n: (0, 0)),
            pl.BlockSpec(W1b.shape, lambda b, n: (0, 0)),
            pl.BlockSpec(b1r.shape, lambda b, n: (0, 0)),
            pl.BlockSpec(W2.shape, lambda b, n: (0, 0)),
            pl.BlockSpec(b2r.shape, lambda b, n: (0, 0)),
        ],
        out_specs=pl.BlockSpec((1, TN, W2.shape[-1]), lambda b, n: (b, n, 0)),
        out_shape=jax.ShapeDtypeStruct((B, N, W2.shape[-1]), jnp.float32),
    )(a1, n1, feat1, a2, n2, feat2, W1a, W1b, b1r, W2, b2r)
    return out
